# baseline (speedup 1.0000x reference)
import jax
import jax.numpy as jnp
from jax.experimental import pallas as pl
from jax.experimental.pallas import tpu as pltpu

_ROW_BLOCK = 256


def _emb_kernel(x_ref, w_ref, out_ref):
    rb, Lp = x_ref.shape
    LD = w_ref.shape[1]
    D = LD // Lp
    x = x_ref[...]
    xc = jnp.where(jnp.isnan(x), jnp.float32(0.0), x)
    xr = jnp.broadcast_to(xc[:, :, None], (rb, Lp, D)).reshape(rb, LD)
    out_ref[...] = xr * w_ref[...]


def kernel(x, emb_weight, cls_token):
    b, L = x.shape
    D = emb_weight.shape[1]
    xa = jnp.concatenate([x, jnp.ones((b, 1), jnp.float32)], axis=1)
    wfull = jnp.concatenate(
        [emb_weight[1 : L + 1].reshape(1, L * D), cls_token.reshape(1, D)], axis=1
    )
    rb = _ROW_BLOCK
    out2d = pl.pallas_call(
        _emb_kernel,
        grid=(b // rb,),
        in_specs=[
            pl.BlockSpec((rb, L + 1), lambda i: (i, 0)),
            pl.BlockSpec((1, (L + 1) * D), lambda i: (0, 0)),
        ],
        out_specs=pl.BlockSpec((rb, (L + 1) * D), lambda i: (i, 0)),
        out_shape=jax.ShapeDtypeStruct((b, (L + 1) * D), jnp.float32),
        compiler_params=pltpu.CompilerParams(
            dimension_semantics=("parallel",),
        ),
    )(xa, wfull)
    return out2d.reshape(b, L + 1, D)


# speedup vs baseline: 1.4439x; 1.4439x over previous
import jax
import jax.numpy as jnp
from jax.experimental import pallas as pl
from jax.experimental.pallas import tpu as pltpu

_ROW_BLOCK = 256


def _emb_kernel(x_ref, m_ref, out_ref):
    x = x_ref[...]
    xc = jnp.where(jnp.isnan(x), jnp.float32(0.0), x)
    out_ref[...] = jnp.dot(xc, m_ref[...], preferred_element_type=jnp.float32)


def kernel(x, emb_weight, cls_token):
    b, L = x.shape
    D = emb_weight.shape[1]
    xa = jnp.concatenate([x, jnp.ones((b, 1), jnp.float32)], axis=1)
    wrows = jnp.concatenate([emb_weight[1 : L + 1], cls_token.reshape(1, D)], axis=0)
    sel = (
        jnp.arange(L + 1, dtype=jnp.int32)[:, None]
        == (jnp.arange((L + 1) * D, dtype=jnp.int32) // D)[None, :]
    )
    m = jnp.where(
        sel, jnp.tile(wrows.reshape(1, (L + 1) * D), (L + 1, 1)), jnp.float32(0.0)
    )
    rb = _ROW_BLOCK
    out2d = pl.pallas_call(
        _emb_kernel,
        grid=(b // rb,),
        in_specs=[
            pl.BlockSpec((rb, L + 1), lambda i: (i, 0)),
            pl.BlockSpec((L + 1, (L + 1) * D), lambda i: (0, 0)),
        ],
        out_specs=pl.BlockSpec((rb, (L + 1) * D), lambda i: (i, 0)),
        out_shape=jax.ShapeDtypeStruct((b, (L + 1) * D), jnp.float32),
        compiler_params=pltpu.CompilerParams(
            dimension_semantics=("parallel",),
        ),
    )(xa, m)
    return out2d.reshape(b, L + 1, D)


# speedup vs baseline: 1.4454x; 1.0010x over previous
import jax
import jax.numpy as jnp
from jax.experimental import pallas as pl
from jax.experimental.pallas import tpu as pltpu

_ROW_BLOCK = 128


def _emb_kernel(x_ref, m_ref, out_ref):
    x = x_ref[...]
    xc = jnp.where(jnp.isnan(x), jnp.float32(0.0), x)
    out_ref[...] = jnp.dot(xc, m_ref[...], preferred_element_type=jnp.float32)


def kernel(x, emb_weight, cls_token):
    b, L = x.shape
    D = emb_weight.shape[1]
    xa = jnp.concatenate([x, jnp.ones((b, 1), jnp.float32)], axis=1)
    wrows = jnp.concatenate([emb_weight[1 : L + 1], cls_token.reshape(1, D)], axis=0)
    sel = (
        jnp.arange(L + 1, dtype=jnp.int32)[:, None]
        == (jnp.arange((L + 1) * D, dtype=jnp.int32) // D)[None, :]
    )
    m = jnp.where(
        sel, jnp.tile(wrows.reshape(1, (L + 1) * D), (L + 1, 1)), jnp.float32(0.0)
    )
    rb = _ROW_BLOCK
    out2d = pl.pallas_call(
        _emb_kernel,
        grid=(b // rb,),
        in_specs=[
            pl.BlockSpec((rb, L + 1), lambda i: (i, 0)),
            pl.BlockSpec((L + 1, (L + 1) * D), lambda i: (0, 0)),
        ],
        out_specs=pl.BlockSpec((rb, (L + 1) * D), lambda i: (i, 0)),
        out_shape=jax.ShapeDtypeStruct((b, (L + 1) * D), jnp.float32),
        compiler_params=pltpu.CompilerParams(
            dimension_semantics=("parallel",),
        ),
    )(xa, m)
    return out2d.reshape(b, L + 1, D)


# speedup vs baseline: 1.4614x; 1.0111x over previous
import jax
import jax.numpy as jnp
from jax.experimental import pallas as pl
from jax.experimental.pallas import tpu as pltpu

_ROW_BLOCK = 256
_N_CHUNKS = 4


def _chunk_bounds(Lp):
    step = -(-Lp // _N_CHUNKS)
    step += step % 2
    edges = list(range(0, Lp, step)) + [Lp]
    return list(zip(edges[:-1], edges[1:]))


def _emb_kernel(x_ref, *refs):
    m_refs, out_ref = refs[:-1], refs[-1]
    Lp = x_ref.shape[1]
    D = out_ref.shape[1] // Lp
    x = x_ref[...]
    xc = jnp.where(jnp.isnan(x), jnp.float32(0.0), x)
    for (l0, l1), m_ref in zip(_chunk_bounds(Lp), m_refs):
        out_ref[:, l0 * D : l1 * D] = jnp.dot(
            xc[:, l0:l1], m_ref[...], preferred_element_type=jnp.float32
        )


def kernel(x, emb_weight, cls_token):
    b, L = x.shape
    D = emb_weight.shape[1]
    xa = jnp.concatenate([x, jnp.ones((b, 1), jnp.float32)], axis=1)
    wrows = jnp.concatenate([emb_weight[1 : L + 1], cls_token.reshape(1, D)], axis=0)
    bounds = _chunk_bounds(L + 1)
    ms = []
    for l0, l1 in bounds:
        k = l1 - l0
        sel = (
            jnp.arange(k, dtype=jnp.int32)[:, None]
            == (jnp.arange(k * D, dtype=jnp.int32) // D)[None, :]
        )
        wc = wrows[l0:l1].reshape(1, k * D)
        ms.append(jnp.where(sel, jnp.tile(wc, (k, 1)), jnp.float32(0.0)))
    rb = _ROW_BLOCK
    m_specs = [
        pl.BlockSpec(m.shape, lambda i: (0, 0)) for m in ms
    ]
    out2d = pl.pallas_call(
        _emb_kernel,
        grid=(b // rb,),
        in_specs=[pl.BlockSpec((rb, L + 1), lambda i: (i, 0))] + m_specs,
        out_specs=pl.BlockSpec((rb, (L + 1) * D), lambda i: (i, 0)),
        out_shape=jax.ShapeDtypeStruct((b, (L + 1) * D), jnp.float32),
        compiler_params=pltpu.CompilerParams(
            dimension_semantics=("parallel",),
        ),
    )(xa, *ms)
    return out2d.reshape(b, L + 1, D)
